# baseline (speedup 1.0000x reference)
import functools

import jax
import jax.numpy as jnp
from jax import lax
from jax.experimental import pallas as pl
from jax.experimental.pallas import tpu as pltpu
from jax.experimental.pallas import tpu_sc as plsc

B = 16384
K = 32
NC = 2
NS = 16
NW = NC * NS
BPW = B // NW
CH = 128
NCH = BPW // CH
L = 16
VR = 128


_mesh = plsc.VectorSubcoreMesh(core_axis_name="c", subcore_axis_name="s")


@functools.partial(
    pl.kernel,
    mesh=_mesh,
    compiler_params=pltpu.CompilerParams(needs_layout_passes=False),
    out_type=jax.ShapeDtypeStruct((B,), jnp.float32),
    scratch_types=[
        pltpu.VMEM((BPW,), jnp.int32),
        pltpu.VMEM((BPW,), jnp.int32),
        pltpu.VMEM((BPW,), jnp.int32),
        pltpu.VMEM((BPW,), jnp.int32),
        pltpu.VMEM((CH, VR), jnp.float32),
        pltpu.VMEM((CH, VR), jnp.float32),
        pltpu.VMEM((BPW,), jnp.float32),
        pltpu.SemaphoreType.DMA,
    ],
)
def _pmf_sc(user_hbm, item_hbm, wu_v_hbm, wi_v_hbm, out_hbm,
            uidx, iidx, urow, irow, ublk, iblk, oacc, sem):
    wid = lax.axis_index("s") * NC + lax.axis_index("c")
    base = wid * BPW

    pltpu.sync_copy(user_hbm.at[pl.ds(base, BPW)], uidx)
    pltpu.sync_copy(item_hbm.at[pl.ds(base, BPW)], iidx)

    def rows(j, carry):
        sl = pl.ds(j * L, L)
        urow[sl] = lax.shift_right_logical(uidx[sl], 2)
        irow[sl] = lax.shift_right_logical(iidx[sl], 2)
        return carry

    lax.fori_loop(0, BPW // L, rows, 0)

    def chunk(c, carry):
        csl = pl.ds(c * CH, CH)
        cu = pltpu.async_copy(wu_v_hbm.at[urow.at[csl]], ublk, sem)
        ci = pltpu.async_copy(wi_v_hbm.at[irow.at[csl]], iblk, sem)
        cu.wait()
        ci.wait()

        def group(g, inner):
            gsl = pl.ds(c * CH + g * L, L)
            rid = g * L + lax.iota(jnp.int32, L)
            uoff = lax.shift_left(jnp.bitwise_and(uidx[gsl], 3), 5)
            ioff = lax.shift_left(jnp.bitwise_and(iidx[gsl], 3), 5)
            acc = jnp.zeros((L,), jnp.float32)
            for k in range(K):
                u = plsc.load_gather(ublk, [rid, uoff + k])
                v = plsc.load_gather(iblk, [rid, ioff + k])
                acc = acc + u * v
            oacc[gsl] = acc
            return inner

        lax.fori_loop(0, CH // L, group, 0)
        return carry

    lax.fori_loop(0, NCH, chunk, 0)

    pltpu.sync_copy(oacc, out_hbm.at[pl.ds(base, BPW)])


def kernel(user, item, W_user, W_item):
    wu_v = W_user.reshape(250000, 128)
    wi_v = W_item.reshape(250000, 128)
    return _pmf_sc(user, item, wu_v, wi_v)
